# Initial kernel scaffold; baseline (speedup 1.0000x reference)
import jax
import jax.numpy as jnp
from jax.experimental import pallas as pl


def kernel(features, scribbles, batched_fg_coords_list, batched_bg_coords_list, random_bg_queries):
    raise NotImplementedError("write your pallas kernel here")


# speedup vs baseline: 1.0668x; 1.0668x over previous
import functools

import jax
import jax.numpy as jnp
from jax.experimental import pallas as pl


def _prep_kernel(hw, num_levels, m_ref, o_ref):
    m = m_ref[...]
    sel = (m > 0.5).astype(jnp.float32)
    counts = jnp.sum(sel, axis=-1, keepdims=True)
    iota = jax.lax.broadcasted_iota(jnp.int32, m.shape, 2)
    maxv = jnp.max(m, axis=-1, keepdims=True)
    cand = jnp.where(m == maxv, iota, hw)
    amax = jnp.min(cand, axis=-1, keepdims=True)
    onehot = (iota == amax).astype(jnp.float32)
    sel_eff = jnp.where(counts > 0.0, sel, onehot)
    scale = 1.0 / (num_levels * jnp.maximum(counts, 1.0))
    o_ref[...] = sel_eff * scale


def _pool_kernel(f_ref, s_ref, o_ref):
    l = pl.program_id(1)
    k = pl.program_id(2)
    f = f_ref[0, 0]
    s = s_ref[0]
    part = jax.lax.dot_general(
        s, f, (((1,), (1,)), ((), ())), preferred_element_type=jnp.float32
    )

    @pl.when((l == 0) & (k == 0))
    def _init():
        o_ref[0] = part

    @pl.when((l > 0) | (k > 0))
    def _acc():
        o_ref[0] += part


def kernel(features, scribbles, batched_fg_coords_list, batched_bg_coords_list,
           random_bg_queries):
    L, B, C, H, W = features.shape
    I = scribbles.shape[1]
    HW = H * W
    fmap = features.reshape(L, B, C, HW)
    m = scribbles.astype(jnp.float32).reshape(B, I, HW)

    sel_scaled = pl.pallas_call(
        functools.partial(_prep_kernel, HW, L),
        out_shape=jax.ShapeDtypeStruct((B, I, HW), jnp.float32),
    )(m)

    hwc = 2048
    k_steps = HW // hwc
    out = pl.pallas_call(
        _pool_kernel,
        grid=(B, L, k_steps),
        in_specs=[
            pl.BlockSpec((1, 1, C, hwc), lambda b, l, k: (l, b, 0, k)),
            pl.BlockSpec((1, I, hwc), lambda b, l, k: (b, 0, k)),
        ],
        out_specs=pl.BlockSpec((1, I, C), lambda b, l, k: (b, 0, 0)),
        out_shape=jax.ShapeDtypeStruct((B, I, C), jnp.float32),
    )(fmap, sel_scaled)

    return out[:, None, :, :]


# speedup vs baseline: 1.1596x; 1.0870x over previous
import functools

import jax
import jax.numpy as jnp
from jax.experimental import pallas as pl


def _prep_kernel(hw, num_levels, m_ref, o_ref):
    m = m_ref[...]
    sel = (m > 0.5).astype(jnp.float32)
    counts = jnp.sum(sel, axis=-1, keepdims=True)
    iota = jax.lax.broadcasted_iota(jnp.int32, m.shape, 2)
    maxv = jnp.max(m, axis=-1, keepdims=True)
    cand = jnp.where(m == maxv, iota, hw)
    amax = jnp.min(cand, axis=-1, keepdims=True)
    onehot = (iota == amax).astype(jnp.float32)
    sel_eff = jnp.where(counts > 0.0, sel, onehot)
    scale = 1.0 / (num_levels * jnp.maximum(counts, 1.0))
    o_ref[...] = sel_eff * scale


def _pool_kernel(f_ref, s_ref, o_ref):
    f = f_ref[0, 0] + f_ref[1, 0]
    s = s_ref[0]
    o_ref[0, 0] = jax.lax.dot_general(
        s, f, (((1,), (1,)), ((), ())), preferred_element_type=jnp.float32
    )


def kernel(features, scribbles, batched_fg_coords_list, batched_bg_coords_list,
           random_bg_queries):
    L, B, C, H, W = features.shape
    I = scribbles.shape[1]
    HW = H * W
    fmap = features.reshape(L, B, C, HW)
    m = scribbles.astype(jnp.float32).reshape(B, I, HW)

    sel_scaled = pl.pallas_call(
        functools.partial(_prep_kernel, HW, L),
        out_shape=jax.ShapeDtypeStruct((B, I, HW), jnp.float32),
    )(m)

    cc = 32
    k_steps = C // cc
    out = pl.pallas_call(
        _pool_kernel,
        grid=(B, k_steps),
        in_specs=[
            pl.BlockSpec((L, 1, cc, HW), lambda b, k: (0, b, k, 0)),
            pl.BlockSpec((1, I, HW), lambda b, k: (b, 0, 0)),
        ],
        out_specs=pl.BlockSpec((1, 1, I, cc), lambda b, k: (b, k, 0, 0)),
        out_shape=jax.ShapeDtypeStruct((B, k_steps, I, cc), jnp.float32),
    )(fmap, sel_scaled)

    out = jnp.transpose(out, (0, 2, 1, 3)).reshape(B, I, C)
    return out[:, None, :, :]


# speedup vs baseline: 1.1774x; 1.0154x over previous
import functools

import jax
import jax.numpy as jnp
from jax.experimental import pallas as pl


def _prep_kernel(hw, num_levels, m_ref, o_ref):
    m = m_ref[...]
    sel = (m > 0.5).astype(jnp.float32)
    counts = jnp.sum(sel, axis=-1, keepdims=True)
    iota = jax.lax.broadcasted_iota(jnp.int32, m.shape, 2)
    maxv = jnp.max(m, axis=-1, keepdims=True)
    cand = jnp.where(m == maxv, iota, hw)
    amax = jnp.min(cand, axis=-1, keepdims=True)
    onehot = (iota == amax).astype(jnp.float32)
    sel_eff = jnp.where(counts > 0.0, sel, onehot)
    scale = 1.0 / (num_levels * jnp.maximum(counts, 1.0))
    o_ref[...] = sel_eff * scale


def _pool_kernel(cg, f0_ref, f1_ref, f2_ref, f3_ref, s_ref, o_ref):
    s = s_ref[0]
    for g, fr in enumerate((f0_ref, f1_ref, f2_ref, f3_ref)):
        f = fr[0, 0] + fr[1, 0]
        o_ref[0, 0, :, g * cg:(g + 1) * cg] = jax.lax.dot_general(
            s, f, (((1,), (1,)), ((), ())), preferred_element_type=jnp.float32
        )


def kernel(features, scribbles, batched_fg_coords_list, batched_bg_coords_list,
           random_bg_queries):
    L, B, C, H, W = features.shape
    I = scribbles.shape[1]
    HW = H * W
    fmap = features.reshape(L, B, C, HW)
    m = scribbles.astype(jnp.float32).reshape(B, I, HW)

    sel_scaled = pl.pallas_call(
        functools.partial(_prep_kernel, HW, L),
        out_shape=jax.ShapeDtypeStruct((B, I, HW), jnp.float32),
    )(m)

    cc = 64
    G = 4
    cg = cc // G
    k_steps = C // cc

    def f_spec(g):
        return pl.BlockSpec((L, 1, cg, HW), lambda b, k, g=g: (0, b, k * G + g, 0))

    out = pl.pallas_call(
        functools.partial(_pool_kernel, cg),
        grid=(B, k_steps),
        in_specs=[f_spec(0), f_spec(1), f_spec(2), f_spec(3),
                  pl.BlockSpec((1, I, HW), lambda b, k: (b, 0, 0))],
        out_specs=pl.BlockSpec((1, 1, I, cc), lambda b, k: (b, k, 0, 0)),
        out_shape=jax.ShapeDtypeStruct((B, k_steps, I, cc), jnp.float32),
    )(fmap, fmap, fmap, fmap, sel_scaled)

    out = jnp.transpose(out, (0, 2, 1, 3)).reshape(B, I, C)
    return out[:, None, :, :]


# speedup vs baseline: 1.2053x; 1.0237x over previous
import functools

import jax
import jax.numpy as jnp
from jax.experimental import pallas as pl
from jax.experimental.pallas import tpu as pltpu


def _fused_kernel(num_levels, cc, nbuf, f_hbm, m_ref, o_ref, buf, sel, sem):
    L = num_levels
    B, I, HW = m_ref.shape
    KC = o_ref.shape[1]
    T = B * KC

    def issue(t):
        b = t // KC
        kc = t % KC
        slot = jax.lax.rem(t, nbuf)
        pltpu.make_async_copy(
            f_hbm.at[:, b, pl.ds(kc * cc, cc), :], buf.at[slot], sem.at[slot]
        ).start()

    for t in range(nbuf):
        issue(t)

    m = m_ref[...]
    s01 = (m > 0.5).astype(jnp.float32)
    counts = jnp.sum(s01, axis=-1, keepdims=True)
    iota = jax.lax.broadcasted_iota(jnp.int32, m.shape, 2)
    maxv = jnp.max(m, axis=-1, keepdims=True)
    amax = jnp.min(jnp.where(m == maxv, iota, HW), axis=-1, keepdims=True)
    onehot = (iota == amax).astype(jnp.float32)
    sel_eff = jnp.where(counts > 0.0, s01, onehot)
    sel[...] = sel_eff * (1.0 / (L * jnp.maximum(counts, 1.0)))

    def body(t, _):
        b = t // KC
        kc = t % KC
        slot = jax.lax.rem(t, nbuf)
        pltpu.make_async_copy(
            f_hbm.at[:, b, pl.ds(kc * cc, cc), :], buf.at[slot], sem.at[slot]
        ).wait()
        f = buf[slot, 0]
        for l in range(1, L):
            f = f + buf[slot, l]
        part = jax.lax.dot_general(
            sel[b], f, (((1,), (1,)), ((), ())),
            preferred_element_type=jnp.float32,
        )
        o_ref[b, kc] = part

        @pl.when(t + nbuf < T)
        def _reissue():
            issue(t + nbuf)

        return None

    jax.lax.fori_loop(0, T, body, None)


def kernel(features, scribbles, batched_fg_coords_list, batched_bg_coords_list,
           random_bg_queries):
    L, B, C, H, W = features.shape
    I = scribbles.shape[1]
    HW = H * W
    fmap = features.reshape(L, B, C, HW)
    m = scribbles.astype(jnp.float32).reshape(B, I, HW)

    cc = 32
    nbuf = 4
    out = pl.pallas_call(
        functools.partial(_fused_kernel, L, cc, nbuf),
        in_specs=[
            pl.BlockSpec(memory_space=pltpu.MemorySpace.HBM),
            pl.BlockSpec(memory_space=pltpu.MemorySpace.VMEM),
        ],
        out_specs=pl.BlockSpec(memory_space=pltpu.MemorySpace.VMEM),
        out_shape=jax.ShapeDtypeStruct((B, C // cc, I, cc), jnp.float32),
        scratch_shapes=[
            pltpu.VMEM((nbuf, L, cc, HW), jnp.float32),
            pltpu.VMEM((B, I, HW), jnp.float32),
            pltpu.SemaphoreType.DMA((nbuf,)),
        ],
    )(fmap, m)

    out = jnp.transpose(out, (0, 2, 1, 3)).reshape(B, I, C)
    return out[:, None, :, :]
